# baseline (speedup 1.0000x reference)
import functools
import math

import jax
import jax.numpy as jnp
from jax import lax
from jax.experimental import pallas as pl

P = 128
K = 16
F = 16
BN_EPS = 1e-3
NEG_SHIFT = 1e9


def _dot(a, b, dims, fast=False):
    return lax.dot_general(
        a, b, (dims, ((), ())),
        precision=(lax.Precision.DEFAULT if fast else None),
        preferred_element_type=jnp.float32)


BT = 16


def _edgeconv_body(feat_ref, *refs):
    out_ref = refs[-1]
    bn0A = refs[0][...]
    bn0B = refs[1][...]
    layer_refs = refs[2:-1]
    W = BT * P

    feat_w = jnp.concatenate([feat_ref[s] for s in range(BT)], axis=1)
    mask = jnp.any(feat_w != 0.0, axis=0, keepdims=True)
    maskf = jnp.where(mask, 1.0, 0.0)
    shift = (1.0 - maskf) * NEG_SHIFT

    fts_w = feat_w * bn0A + bn0B
    f0 = feat_w[0:1, :]
    f1 = feat_w[1:2, :]
    pts_w = jnp.concatenate([f0 * jnp.cos(f1), f0 * jnp.sin(f1)], axis=0)

    cio = lax.broadcasted_iota(jnp.int32, (P, P), 0)

    for l in range(3):
        WdT, WbT, b1, W2T, b2, W3T, b3, scWT, scb = (
            r[...] for r in layer_refs[9 * l:9 * (l + 1)])
        if l > 0:
            pts_w = fts_w + shift
        else:
            pts_w = pts_w + shift
        cdim = pts_w.shape[0]

        cp_w = _dot(WdT, fts_w, ((1,), (0,))) + b1
        if cdim >= 32:
            zz_w = _dot(WbT, fts_w, ((1,), (0,)))
        gs, cps = [], []
        for s in range(BT):
            p_s = pts_w[:, s * P:(s + 1) * P]
            tp = p_s.T
            m = _dot(tp, p_s, ((1,), (0,)))
            r_row = _dot(jnp.ones((1, cdim), jnp.float32), p_s * p_s,
                         ((1,), (0,)))
            r_col = _dot(tp * tp, jnp.ones((cdim, 1), jnp.float32),
                         ((1,), (0,)))
            d = r_col - 2.0 * m + r_row

            ki = lax.bitcast_convert_type(jnp.maximum(d, 0.0), jnp.int32)
            ki = (ki & jnp.int32(-128)) | cio

            ohs = []
            for t in range(K + 1):
                mn = jnp.min(ki, axis=0, keepdims=True)
                oh = ki == mn
                ki = jnp.where(oh, jnp.int32(0x7FFFFFFF), ki)
                if t > 0:
                    ohs.append(jnp.where(oh, 1.0, 0.0))
            O_s = jnp.concatenate(ohs, axis=1)
            if cdim < 32:
                gs.append(_dot(fts_w[:, s * P:(s + 1) * P], O_s,
                               ((1,), (0,)), fast=True))
            else:
                gs.append(_dot(zz_w[:, s * P:(s + 1) * P], O_s,
                               ((1,), (0,)), fast=True))
            cps.append(jnp.concatenate(
                [cp_w[:, s * P:(s + 1) * P]] * K, axis=1))
        G = jnp.concatenate(gs, axis=1)
        if cdim < 32:
            G = _dot(WbT, G, ((1,), (0,)), fast=True)
        CP = jnp.concatenate(cps, axis=1)
        x = jnp.maximum(G + CP, 0.0)
        x = jnp.maximum(_dot(W2T, x, ((1,), (0,)), fast=True) + b2, 0.0)
        x = jnp.maximum(_dot(W3T, x, ((1,), (0,)), fast=True) + b3, 0.0)

        fo = []
        for s in range(BT):
            xs = x[:, s * K * P:(s + 1) * K * P]
            acc = xs[:, 0:P]
            for k in range(1, K):
                acc = acc + xs[:, k * P:(k + 1) * P]
            fo.append(acc)
        fts_out = jnp.concatenate(fo, axis=1) * (1.0 / K)

        sc = _dot(scWT, fts_w, ((1,), (0,))) + scb
        fts_w = jnp.maximum(sc + fts_out, 0.0)

    sio = lax.broadcasted_iota(jnp.int32, (W, BT), 0)
    bio = lax.broadcasted_iota(jnp.int32, (W, BT), 1)
    ones_bd = jnp.where(sio // P == bio, 1.0, 0.0)
    pool = _dot(fts_w * maskf, ones_bd, ((1,), (0,))) * (1.0 / P)
    for s in range(BT):
        out_ref[s] = pool[:, s:s + 1]


def _head_body(pool_ref, d1W_ref, d1b_ref, d2W_ref, d2b_ref, out_ref):
    x = jnp.maximum(
        _dot(pool_ref[...], d1W_ref[...], ((1,), (0,))) + d1b_ref[...], 0.0)
    lg = _dot(x, d2W_ref[...], ((1,), (0,))) + d2b_ref[...]
    mx = jnp.max(lg, axis=1, keepdims=True)
    e = jnp.exp(lg - mx)
    out_ref[...] = e / jnp.sum(e, axis=1, keepdims=True)


@functools.partial(jax.jit, static_argnames=("interpret",))
def _run(features, flat_w, head_w, interpret=False):
    B = features.shape[0]

    in_specs = [pl.BlockSpec((BT, F, P), lambda b: (b, 0, 0))]
    for w in flat_w:
        in_specs.append(
            pl.BlockSpec(w.shape, functools.partial(
                lambda nd, b: (0,) * nd, w.ndim)))

    pool = pl.pallas_call(
        _edgeconv_body,
        grid=(B // BT,),
        in_specs=in_specs,
        out_specs=pl.BlockSpec((BT, 32, 1), lambda b: (b, 0, 0)),
        out_shape=jax.ShapeDtypeStruct((B, 32, 1), jnp.float32),
        interpret=interpret,
    )(features, *flat_w)
    pool = pool.reshape(B, 32)

    d1W, d1b, d2W, d2b = head_w
    out = pl.pallas_call(
        _head_body,
        in_specs=[pl.BlockSpec(x.shape, functools.partial(
            lambda nd: (0,) * nd, x.ndim))
            for x in (pool, d1W, d1b, d2W, d2b)],
        out_specs=pl.BlockSpec((B, 10), lambda: (0, 0)),
        out_shape=jax.ShapeDtypeStruct((B, 10), jnp.float32),
        interpret=interpret,
    )(pool, d1W, d1b, d2W, d2b)
    return out


def _prepare(input, params):
    features = jnp.swapaxes(input[0], 1, 2)
    s = 1.0 / math.sqrt(1.0 + BN_EPS)

    flat_w = [
        (s * params['bn0_g']).reshape(F, 1),
        params['bn0_b'].reshape(F, 1).astype(jnp.float32),
    ]
    for p in params['ec']:
        (W1, g1, b1), (W2, g2, b2), (W3, g3, b3) = p['convs']
        cin = W1.shape[0] // 2
        W1e = W1 * (s * g1)[None, :]
        Wt, Wb = W1e[:cin], W1e[cin:]
        flat_w += [
            (Wt - Wb).T, Wb.T, b1.reshape(-1, 1),
            (W2 * (s * g2)[None, :]).T, b2.reshape(-1, 1),
            (W3 * (s * g3)[None, :]).T, b3.reshape(-1, 1),
            (p['sc_W'] * (s * p['sc_g'])[None, :]).T,
            p['sc_b'].reshape(-1, 1),
        ]
    head_w = (params['d1_W'], params['d1_b'].reshape(1, -1),
              params['d2_W'], params['d2_b'].reshape(1, -1))
    return features, tuple(flat_w), head_w


def kernel(input, params):
    features, flat_w, head_w = _prepare(input, params)
    return _run(features, flat_w, head_w)


# speedup vs baseline: 1.6376x; 1.6376x over previous
import functools
import math

import jax
import jax.numpy as jnp
from jax import lax
from jax.experimental import pallas as pl

P = 128
K = 16
F = 16
BN_EPS = 1e-3
NEG_SHIFT = 1e9


def _dot(a, b, dims, fast=False):
    return lax.dot_general(
        a, b, (dims, ((), ())),
        precision=(lax.Precision.DEFAULT if fast else None),
        preferred_element_type=jnp.float32)


BT = 16


def _edgeconv_body(feat_ref, *refs):
    out_ref = refs[-1]
    bn0A = refs[0][...]
    bn0B = refs[1][...]
    layer_refs = refs[2:-1]
    W = BT * P

    feat_w = jnp.concatenate([feat_ref[s] for s in range(BT)], axis=1)
    mask = jnp.any(feat_w != 0.0, axis=0, keepdims=True)
    maskf = jnp.where(mask, 1.0, 0.0)
    shift = (1.0 - maskf) * NEG_SHIFT

    fts_w = feat_w * bn0A + bn0B
    f0 = feat_w[0:1, :]
    f1 = feat_w[1:2, :]
    pts_w = jnp.concatenate([f0 * jnp.cos(f1), f0 * jnp.sin(f1)], axis=0)

    cio = lax.broadcasted_iota(jnp.int32, (P, W), 0)

    for l in range(3):
        WdT, WbT, b1, W2T, b2, W3T, b3, scWT, scb = (
            r[...] for r in layer_refs[9 * l:9 * (l + 1)])
        if l > 0:
            pts_w = fts_w + shift
        else:
            pts_w = pts_w + shift
        cdim = pts_w.shape[0]

        ds = []
        for s in range(BT):
            p_s = pts_w[:, s * P:(s + 1) * P]
            tp = p_s.T
            m = _dot(tp, p_s, ((1,), (0,)))
            r_row = _dot(jnp.ones((1, cdim), jnp.float32), p_s * p_s,
                         ((1,), (0,)))
            r_col = _dot(tp * tp, jnp.ones((cdim, 1), jnp.float32),
                         ((1,), (0,)))
            ds.append(r_col - 2.0 * m + r_row)
        d = jnp.concatenate(ds, axis=1)

        ki = lax.bitcast_convert_type(jnp.maximum(d, 0.0), jnp.int32)
        ki = (ki & jnp.int32(-128)) | cio

        ohs = []
        for t in range(K + 1):
            mn = jnp.min(ki, axis=0, keepdims=True)
            oh = ki == mn
            ki = jnp.where(oh, jnp.int32(0x7FFFFFFF), ki)
            if t > 0:
                ohs.append(jnp.where(oh, 1.0, 0.0).astype(jnp.bfloat16))

        cp_w = _dot(WdT, fts_w, ((1,), (0,))) + b1
        if cdim >= 32:
            zz_w = _dot(WbT, fts_w, ((1,), (0,)))
        gs, cps = [], []
        for s in range(BT):
            O_s = jnp.concatenate(
                [oh[:, s * P:(s + 1) * P] for oh in ohs], axis=1)
            if cdim < 32:
                gs.append(_dot(fts_w[:, s * P:(s + 1) * P].astype(jnp.bfloat16), O_s,
                               ((1,), (0,)), fast=True))
            else:
                gs.append(_dot(zz_w[:, s * P:(s + 1) * P].astype(jnp.bfloat16), O_s,
                               ((1,), (0,)), fast=True))
            cps.append(jnp.concatenate(
                [cp_w[:, s * P:(s + 1) * P]] * K, axis=1))
        G = jnp.concatenate(gs, axis=1)
        if cdim < 32:
            G = _dot(WbT.astype(jnp.bfloat16), G.astype(jnp.bfloat16), ((1,), (0,)), fast=True)
        CP = jnp.concatenate(cps, axis=1)
        x = jnp.maximum(G + CP, 0.0)
        x = jnp.maximum(_dot(W2T.astype(jnp.bfloat16), x.astype(jnp.bfloat16), ((1,), (0,)), fast=True) + b2, 0.0)
        x = jnp.maximum(_dot(W3T.astype(jnp.bfloat16), x.astype(jnp.bfloat16), ((1,), (0,)), fast=True) + b3, 0.0)

        fo = []
        for s in range(BT):
            xs = x[:, s * K * P:(s + 1) * K * P]
            acc = xs[:, 0:P]
            for k in range(1, K):
                acc = acc + xs[:, k * P:(k + 1) * P]
            fo.append(acc)
        fts_out = jnp.concatenate(fo, axis=1) * (1.0 / K)

        sc = _dot(scWT, fts_w, ((1,), (0,))) + scb
        fts_w = jnp.maximum(sc + fts_out, 0.0)

    sio = lax.broadcasted_iota(jnp.int32, (W, BT), 0)
    bio = lax.broadcasted_iota(jnp.int32, (W, BT), 1)
    ones_bd = jnp.where(sio // P == bio, 1.0, 0.0)
    pool = _dot(fts_w * maskf, ones_bd, ((1,), (0,))) * (1.0 / P)
    for s in range(BT):
        out_ref[s] = pool[:, s:s + 1]


def _head_body(pool_ref, d1W_ref, d1b_ref, d2W_ref, d2b_ref, out_ref):
    x = jnp.maximum(
        _dot(pool_ref[...], d1W_ref[...], ((1,), (0,))) + d1b_ref[...], 0.0)
    lg = _dot(x, d2W_ref[...], ((1,), (0,))) + d2b_ref[...]
    mx = jnp.max(lg, axis=1, keepdims=True)
    e = jnp.exp(lg - mx)
    out_ref[...] = e / jnp.sum(e, axis=1, keepdims=True)


@functools.partial(jax.jit, static_argnames=("interpret",))
def _run(features, flat_w, head_w, interpret=False):
    B = features.shape[0]

    in_specs = [pl.BlockSpec((BT, F, P), lambda b: (b, 0, 0))]
    for w in flat_w:
        in_specs.append(
            pl.BlockSpec(w.shape, functools.partial(
                lambda nd, b: (0,) * nd, w.ndim)))

    pool = pl.pallas_call(
        _edgeconv_body,
        grid=(B // BT,),
        in_specs=in_specs,
        out_specs=pl.BlockSpec((BT, 32, 1), lambda b: (b, 0, 0)),
        out_shape=jax.ShapeDtypeStruct((B, 32, 1), jnp.float32),
        interpret=interpret,
    )(features, *flat_w)
    pool = pool.reshape(B, 32)

    d1W, d1b, d2W, d2b = head_w
    out = pl.pallas_call(
        _head_body,
        in_specs=[pl.BlockSpec(x.shape, functools.partial(
            lambda nd: (0,) * nd, x.ndim))
            for x in (pool, d1W, d1b, d2W, d2b)],
        out_specs=pl.BlockSpec((B, 10), lambda: (0, 0)),
        out_shape=jax.ShapeDtypeStruct((B, 10), jnp.float32),
        interpret=interpret,
    )(pool, d1W, d1b, d2W, d2b)
    return out


def _prepare(input, params):
    features = jnp.swapaxes(input[0], 1, 2)
    s = 1.0 / math.sqrt(1.0 + BN_EPS)

    flat_w = [
        (s * params['bn0_g']).reshape(F, 1),
        params['bn0_b'].reshape(F, 1).astype(jnp.float32),
    ]
    for p in params['ec']:
        (W1, g1, b1), (W2, g2, b2), (W3, g3, b3) = p['convs']
        cin = W1.shape[0] // 2
        W1e = W1 * (s * g1)[None, :]
        Wt, Wb = W1e[:cin], W1e[cin:]
        flat_w += [
            (Wt - Wb).T, Wb.T, b1.reshape(-1, 1),
            (W2 * (s * g2)[None, :]).T, b2.reshape(-1, 1),
            (W3 * (s * g3)[None, :]).T, b3.reshape(-1, 1),
            (p['sc_W'] * (s * p['sc_g'])[None, :]).T,
            p['sc_b'].reshape(-1, 1),
        ]
    head_w = (params['d1_W'], params['d1_b'].reshape(1, -1),
              params['d2_W'], params['d2_b'].reshape(1, -1))
    return features, tuple(flat_w), head_w


def kernel(input, params):
    features, flat_w, head_w = _prepare(input, params)
    return _run(features, flat_w, head_w)


# speedup vs baseline: 1.6526x; 1.0092x over previous
import functools
import math

import jax
import jax.numpy as jnp
from jax import lax
from jax.experimental import pallas as pl

P = 128
K = 16
F = 16
BN_EPS = 1e-3
NEG_SHIFT = 1e9


def _dot(a, b, dims, fast=False):
    return lax.dot_general(
        a, b, (dims, ((), ())),
        precision=(lax.Precision.DEFAULT if fast else None),
        preferred_element_type=jnp.float32)


BT = 16


def _edgeconv_body(feat_ref, *refs):
    out_ref = refs[-1]
    bn0A = refs[0][...]
    bn0B = refs[1][...]
    layer_refs = refs[2:-1]
    W = BT * P

    feat_w = jnp.concatenate([feat_ref[s] for s in range(BT)], axis=1)
    mask = jnp.any(feat_w != 0.0, axis=0, keepdims=True)
    maskf = jnp.where(mask, 1.0, 0.0)
    shift = (1.0 - maskf) * NEG_SHIFT

    fts_w = feat_w * bn0A + bn0B
    f0 = feat_w[0:1, :]
    f1 = feat_w[1:2, :]
    pts_w = jnp.concatenate([f0 * jnp.cos(f1), f0 * jnp.sin(f1)], axis=0)

    cio = lax.broadcasted_iota(jnp.int32, (P, W), 0)

    for l in range(3):
        WdT, WbT, b1, W2T, b2, W3T, b3, scWT, scb = (
            r[...] for r in layer_refs[9 * l:9 * (l + 1)])
        if l > 0:
            pts_w = fts_w + shift
        else:
            pts_w = pts_w + shift
        cdim = pts_w.shape[0]

        ds = []
        for s in range(BT):
            p_s = pts_w[:, s * P:(s + 1) * P]
            tp = p_s.T
            m = _dot(tp, p_s, ((1,), (0,)))
            r_row = _dot(jnp.ones((1, cdim), jnp.float32), p_s * p_s,
                         ((1,), (0,)))
            r_col = _dot(tp * tp, jnp.ones((cdim, 1), jnp.float32),
                         ((1,), (0,)))
            ds.append(r_col - 2.0 * m + r_row)
        d = jnp.concatenate(ds, axis=1)

        ki = lax.bitcast_convert_type(jnp.maximum(d, 0.0), jnp.int32)
        ki = (ki & jnp.int32(-128)) | cio

        ohs = []
        for t in range(K + 1):
            mn = jnp.min(ki, axis=0, keepdims=True)
            oh = ki == mn
            ki = jnp.where(oh, jnp.int32(0x7FFFFFFF), ki)
            if t > 0:
                ohs.append(jnp.where(oh, 1.0, 0.0))

        cp_w = _dot(WdT, fts_w, ((1,), (0,))) + b1
        if cdim >= 32:
            zz_w = _dot(WbT, fts_w, ((1,), (0,)))
        gs, cps = [], []
        for s in range(BT):
            O_s = jnp.concatenate(
                [oh[:, s * P:(s + 1) * P] for oh in ohs], axis=1)
            if cdim < 32:
                gs.append(_dot(fts_w[:, s * P:(s + 1) * P], O_s,
                               ((1,), (0,)), fast=True))
            else:
                gs.append(_dot(zz_w[:, s * P:(s + 1) * P], O_s,
                               ((1,), (0,)), fast=True))
            cps.append(jnp.concatenate(
                [cp_w[:, s * P:(s + 1) * P]] * K, axis=1))
        G = jnp.concatenate(gs, axis=1)
        if cdim < 32:
            G = _dot(WbT, G, ((1,), (0,)), fast=True)
        CP = jnp.concatenate(cps, axis=1)
        x = jnp.maximum(G + CP, 0.0)
        x = jnp.maximum(_dot(W2T, x, ((1,), (0,)), fast=True) + b2, 0.0)
        x = jnp.maximum(_dot(W3T, x, ((1,), (0,)), fast=True) + b3, 0.0)

        fo = []
        for s in range(BT):
            xs = x[:, s * K * P:(s + 1) * K * P]
            acc = xs[:, 0:P]
            for k in range(1, K):
                acc = acc + xs[:, k * P:(k + 1) * P]
            fo.append(acc)
        fts_out = jnp.concatenate(fo, axis=1) * (1.0 / K)

        sc = _dot(scWT, fts_w, ((1,), (0,))) + scb
        fts_w = jnp.maximum(sc + fts_out, 0.0)

    sio = lax.broadcasted_iota(jnp.int32, (W, BT), 0)
    bio = lax.broadcasted_iota(jnp.int32, (W, BT), 1)
    ones_bd = jnp.where(sio // P == bio, 1.0, 0.0)
    pool = _dot(fts_w * maskf, ones_bd, ((1,), (0,))) * (1.0 / P)
    for s in range(BT):
        out_ref[s] = pool[:, s:s + 1]


def _head_body(pool_ref, d1W_ref, d1b_ref, d2W_ref, d2b_ref, out_ref):
    x = jnp.maximum(
        _dot(pool_ref[...], d1W_ref[...], ((1,), (0,))) + d1b_ref[...], 0.0)
    lg = _dot(x, d2W_ref[...], ((1,), (0,))) + d2b_ref[...]
    mx = jnp.max(lg, axis=1, keepdims=True)
    e = jnp.exp(lg - mx)
    out_ref[...] = e / jnp.sum(e, axis=1, keepdims=True)


@functools.partial(jax.jit, static_argnames=("interpret",))
def _run(features, flat_w, head_w, interpret=False):
    B = features.shape[0]

    in_specs = [pl.BlockSpec((BT, F, P), lambda b: (b, 0, 0))]
    for w in flat_w:
        in_specs.append(
            pl.BlockSpec(w.shape, functools.partial(
                lambda nd, b: (0,) * nd, w.ndim)))

    pool = pl.pallas_call(
        _edgeconv_body,
        grid=(B // BT,),
        in_specs=in_specs,
        out_specs=pl.BlockSpec((BT, 32, 1), lambda b: (b, 0, 0)),
        out_shape=jax.ShapeDtypeStruct((B, 32, 1), jnp.float32),
        interpret=interpret,
    )(features, *flat_w)
    pool = pool.reshape(B, 32)

    d1W, d1b, d2W, d2b = head_w
    out = pl.pallas_call(
        _head_body,
        in_specs=[pl.BlockSpec(x.shape, functools.partial(
            lambda nd: (0,) * nd, x.ndim))
            for x in (pool, d1W, d1b, d2W, d2b)],
        out_specs=pl.BlockSpec((B, 10), lambda: (0, 0)),
        out_shape=jax.ShapeDtypeStruct((B, 10), jnp.float32),
        interpret=interpret,
    )(pool, d1W, d1b, d2W, d2b)
    return out


def _prepare(input, params):
    features = jnp.swapaxes(input[0], 1, 2)
    s = 1.0 / math.sqrt(1.0 + BN_EPS)

    flat_w = [
        (s * params['bn0_g']).reshape(F, 1),
        params['bn0_b'].reshape(F, 1).astype(jnp.float32),
    ]
    for p in params['ec']:
        (W1, g1, b1), (W2, g2, b2), (W3, g3, b3) = p['convs']
        cin = W1.shape[0] // 2
        W1e = W1 * (s * g1)[None, :]
        Wt, Wb = W1e[:cin], W1e[cin:]
        flat_w += [
            (Wt - Wb).T, Wb.T, b1.reshape(-1, 1),
            (W2 * (s * g2)[None, :]).T, b2.reshape(-1, 1),
            (W3 * (s * g3)[None, :]).T, b3.reshape(-1, 1),
            (p['sc_W'] * (s * p['sc_g'])[None, :]).T,
            p['sc_b'].reshape(-1, 1),
        ]
    head_w = (params['d1_W'], params['d1_b'].reshape(1, -1),
              params['d2_W'], params['d2_b'].reshape(1, -1))
    return features, tuple(flat_w), head_w


def kernel(input, params):
    features, flat_w, head_w = _prepare(input, params)
    return _run(features, flat_w, head_w)
